# Initial kernel scaffold; baseline (speedup 1.0000x reference)
import jax
import jax.numpy as jnp
from jax.experimental import pallas as pl


def kernel(image, w1, b1, w2, b2, w3, b3, w4, b4, w5, b5, w6, b6, w_out, b_out):
    raise NotImplementedError("write your pallas kernel here")


# speedup vs baseline: 1.9942x; 1.9942x over previous
import functools

import jax
import jax.numpy as jnp
from jax.experimental import pallas as pl

BF16 = jnp.bfloat16
F32 = jnp.float32


def _dot(a, b):
    return jax.lax.dot_general(a, b, (((1,), (0,)), ((), ())),
                               preferred_element_type=F32)


def _shift3(win, W):
    z = jnp.zeros(win.shape[:2] + (1,), win.dtype)
    return [jnp.concatenate([z, win[:, :, :W - 1]], axis=2),
            win,
            jnp.concatenate([win[:, :, 1:], z], axis=2)]


def _accum_rows(srcs, o_ref, bias, relu, TR, Cout, W, out_dtype):
    for r in range(TR):
        acc = jnp.zeros((Cout, W), F32)
        for shifts, w in srcs:
            Cin = shifts[1].shape[1]
            for kx in range(3):
                rhs = shifts[kx][r:r + 3].reshape(3 * Cin, W)
                acc = acc + _dot(w[kx], rhs)
        acc = acc + bias
        if relu:
            acc = jnp.maximum(acc, 0.0)
        o_ref[0, r] = acc.astype(out_dtype)


def _conv_s1_body(*refs, nin, TR, W, Cout, relu, out_dtype):
    o_ref = refs[-1]
    b_ref = refs[-2]
    srcs = []
    for k in range(nin):
        top, cur, bot = refs[3 * k], refs[3 * k + 1], refs[3 * k + 2]
        win = jnp.concatenate([top[0], cur[0], bot[0]], axis=0)
        srcs.append((_shift3(win, W), refs[3 * nin + k]))
    _accum_rows(srcs, o_ref, b_ref[...], relu, TR, Cout, W, out_dtype)


def _halos(x, TR):
    B, H, C, W = x.shape
    z = jnp.zeros((B, 1, C, W), x.dtype)
    top = jnp.concatenate([z, x[:, TR - 1::TR][:, :-1]], axis=1)
    bot = jnp.concatenate([x[:, TR::TR], z], axis=1)
    return top, bot


def _conv_s1(xs, ws, b, relu, TR, out_dtype=BF16):
    B, H, _, W = xs[0].shape
    Cout = ws[0].shape[1]
    nb = H // TR
    args, in_specs = [], []
    for x in xs:
        Cin = x.shape[2]
        top, bot = _halos(x, TR)
        args += [top, x, bot]
        in_specs += [
            pl.BlockSpec((1, 1, Cin, W), lambda bb, i: (bb, i, 0, 0)),
            pl.BlockSpec((1, TR, Cin, W), lambda bb, i: (bb, i, 0, 0)),
            pl.BlockSpec((1, 1, Cin, W), lambda bb, i: (bb, i, 0, 0)),
        ]
    for w in ws:
        args.append(w)
        in_specs.append(pl.BlockSpec(w.shape, lambda bb, i: (0, 0, 0)))
    args.append(b)
    in_specs.append(pl.BlockSpec(b.shape, lambda bb, i: (0, 0)))
    return pl.pallas_call(
        functools.partial(_conv_s1_body, nin=len(xs), TR=TR, W=W, Cout=Cout,
                          relu=relu, out_dtype=out_dtype),
        grid=(B, nb),
        in_specs=in_specs,
        out_specs=pl.BlockSpec((1, TR, Cout, W), lambda bb, i: (bb, i, 0, 0)),
        out_shape=jax.ShapeDtypeStruct((B, H, Cout, W), out_dtype),
    )(*args)


def _conv_s2_body(ec, eb, oc, ob, w, b_ref, o_ref, *, TR, Wo, Cin, Cout):
    z = jnp.zeros((1, Cin, Wo), BF16)
    wine = jnp.concatenate([ec[0], eb[0], z], axis=0)
    wino = jnp.concatenate([oc[0], ob[0], z], axis=0)
    zc = jnp.zeros((2 * TR + 2, Cin, 1), BF16)
    srcs = [wine, wino, jnp.concatenate([wine[:, :, 1:], zc], axis=2)]
    for r in range(TR):
        acc = jnp.zeros((Cout, Wo), F32)
        for kx in range(3):
            rhs = srcs[kx][2 * r:2 * r + 3].reshape(3 * Cin, Wo)
            acc = acc + _dot(w[kx], rhs)
        acc = jnp.maximum(acc + b_ref[...], 0.0)
        o_ref[0, r] = acc.astype(BF16)


def _conv_s2(xe, xo, w, b, TR):
    B, H, Cin, Wo = xe.shape
    Cout = w.shape[1]
    Ho = H // 2
    nb = Ho // TR
    zb = jnp.zeros((B, 1, Cin, Wo), BF16)
    bot_e = jnp.concatenate([xe[:, 2 * TR::2 * TR], zb], axis=1)
    bot_o = jnp.concatenate([xo[:, 2 * TR::2 * TR], zb], axis=1)
    big = lambda: pl.BlockSpec((1, 2 * TR, Cin, Wo), lambda bb, i: (bb, i, 0, 0))
    one = lambda: pl.BlockSpec((1, 1, Cin, Wo), lambda bb, i: (bb, i, 0, 0))
    return pl.pallas_call(
        functools.partial(_conv_s2_body, TR=TR, Wo=Wo, Cin=Cin, Cout=Cout),
        grid=(B, nb),
        in_specs=[big(), one(), big(), one(),
                  pl.BlockSpec(w.shape, lambda bb, i: (0, 0, 0)),
                  pl.BlockSpec(b.shape, lambda bb, i: (0, 0))],
        out_specs=pl.BlockSpec((1, TR, Cout, Wo), lambda bb, i: (bb, i, 0, 0)),
        out_shape=jax.ShapeDtypeStruct((B, Ho, Cout, Wo), BF16),
    )(xe, bot_e, xo, bot_o, w, b)


def _wk(w):
    O, I = w.shape[0], w.shape[1]
    return jnp.transpose(w, (3, 0, 2, 1)).reshape(3, O, 3 * I).astype(BF16)


def kernel(image, w1, b1, w2, b2, w3, b3, w4, b4, w5, b5, w6, b6, w_out, b_out):
    x = jnp.transpose(image, (0, 2, 1, 3)).astype(BF16)
    bb = lambda b: b.reshape(-1, 1).astype(F32)
    t = _conv_s1([x], [_wk(w1)], bb(b1), True, 16)
    e1 = _conv_s1([t], [_wk(w2)], bb(b2), True, 16)
    e1e, e1o = e1[:, :, :, 0::2], e1[:, :, :, 1::2]
    d = _conv_s2(e1e, e1o, _wk(w3), bb(b3), 16)
    d = _conv_s1([d], [_wk(w4)], bb(b4), True, 16)
    u = jnp.repeat(jnp.repeat(d, 2, axis=1), 2, axis=3)
    u = _conv_s1([u], [_wk(w5)], bb(b5), True, 16)
    f = _conv_s1([u, e1], [_wk(w6[:, :32]), _wk(w6[:, 32:])], bb(b6), True, 16)
    y = _conv_s1([f], [_wk(w_out)], bb(b_out), True, 16, out_dtype=F32)
    return jnp.transpose(y, (0, 2, 1, 3))


# speedup vs baseline: 6.4309x; 3.2248x over previous
import functools

import jax
import jax.numpy as jnp
from jax.experimental import pallas as pl

BF16 = jnp.bfloat16
F32 = jnp.float32


def _dot(a, b):
    return jax.lax.dot_general(a, b, (((1,), (0,)), ((), ())),
                               preferred_element_type=F32)


def _shift3(win, W):
    z = jnp.zeros(win.shape[:2] + (1,), win.dtype)
    return [jnp.concatenate([z, win[:, :, :W - 1]], axis=2),
            win,
            jnp.concatenate([win[:, :, 1:], z], axis=2)]


def _accum_rows(srcs, o_ref, bias, relu, TR, Cout, W, out_dtype):
    for r in range(TR):
        acc = jnp.zeros((Cout, W), F32)
        for shifts, w in srcs:
            Cin = shifts[1].shape[1]
            for kx in range(3):
                rhs = shifts[kx][r:r + 3].reshape(3 * Cin, W)
                acc = acc + _dot(w[kx], rhs)
        acc = acc + bias
        if relu:
            acc = jnp.maximum(acc, 0.0)
        o_ref[0, r] = acc.astype(out_dtype)


def _conv_s1_body(*refs, nin, TR, W, Cout, relu, out_dtype):
    o_ref = refs[-1]
    b_ref = refs[-2]
    srcs = []
    for k in range(nin):
        top, cur, bot = refs[3 * k], refs[3 * k + 1], refs[3 * k + 2]
        win = jnp.concatenate([top[0], cur[0], bot[0]], axis=0)
        srcs.append((_shift3(win, W), refs[3 * nin + k]))
    _accum_rows(srcs, o_ref, b_ref[...], relu, TR, Cout, W, out_dtype)


def _halos(x, TR):
    B, H, C, W = x.shape
    z = jnp.zeros((B, 1, C, W), x.dtype)
    top = jnp.concatenate([z, x[:, TR - 1::TR][:, :-1]], axis=1)
    bot = jnp.concatenate([x[:, TR::TR], z], axis=1)
    return top, bot


def _conv_s1(xs, ws, b, relu, TR, out_dtype=BF16):
    B, H, _, W = xs[0].shape
    Cout = ws[0].shape[1]
    nb = H // TR
    args, in_specs = [], []
    for x in xs:
        Cin = x.shape[2]
        top, bot = _halos(x, TR)
        args += [top, x, bot]
        in_specs += [
            pl.BlockSpec((1, 1, Cin, W), lambda bb, i: (bb, i, 0, 0)),
            pl.BlockSpec((1, TR, Cin, W), lambda bb, i: (bb, i, 0, 0)),
            pl.BlockSpec((1, 1, Cin, W), lambda bb, i: (bb, i, 0, 0)),
        ]
    for w in ws:
        args.append(w)
        in_specs.append(pl.BlockSpec(w.shape, lambda bb, i: (0, 0, 0)))
    args.append(b)
    in_specs.append(pl.BlockSpec(b.shape, lambda bb, i: (0, 0)))
    return pl.pallas_call(
        functools.partial(_conv_s1_body, nin=len(xs), TR=TR, W=W, Cout=Cout,
                          relu=relu, out_dtype=out_dtype),
        grid=(B, nb),
        in_specs=in_specs,
        out_specs=pl.BlockSpec((1, TR, Cout, W), lambda bb, i: (bb, i, 0, 0)),
        out_shape=jax.ShapeDtypeStruct((B, H, Cout, W), out_dtype),
    )(*args)


def _conv_s2_body(xc, xb, s0, s1, w, b_ref, o_ref, *, TR, Wo, Cin, Cout):
    W = 2 * Wo
    z = jnp.zeros((1, Cin, W), BF16)
    win = jnp.concatenate([xc[0], xb[0], z], axis=0)
    flat = win.reshape((2 * TR + 2) * Cin, W)
    wine = _dot(flat, s0[...]).astype(BF16).reshape(2 * TR + 2, Cin, Wo)
    wino = _dot(flat, s1[...]).astype(BF16).reshape(2 * TR + 2, Cin, Wo)
    zc = jnp.zeros((2 * TR + 2, Cin, 1), BF16)
    srcs = [wine, wino, jnp.concatenate([wine[:, :, 1:], zc], axis=2)]
    for r in range(TR):
        acc = jnp.zeros((Cout, Wo), F32)
        for kx in range(3):
            rhs = srcs[kx][2 * r:2 * r + 3].reshape(3 * Cin, Wo)
            acc = acc + _dot(w[kx], rhs)
        acc = jnp.maximum(acc + b_ref[...], 0.0)
        o_ref[0, r] = acc.astype(BF16)


def _conv_s2(x, w, b, TR):
    B, H, Cin, W = x.shape
    Cout = w.shape[1]
    Ho, Wo = H // 2, W // 2
    nb = Ho // TR
    zb = jnp.zeros((B, 1, Cin, W), BF16)
    bot = jnp.concatenate([x[:, 2 * TR::2 * TR], zb], axis=1)
    cols = jnp.arange(W)[:, None]
    s0 = (cols == 2 * jnp.arange(Wo)[None, :]).astype(BF16)
    s1 = (cols == 2 * jnp.arange(Wo)[None, :] + 1).astype(BF16)
    return pl.pallas_call(
        functools.partial(_conv_s2_body, TR=TR, Wo=Wo, Cin=Cin, Cout=Cout),
        grid=(B, nb),
        in_specs=[pl.BlockSpec((1, 2 * TR, Cin, W), lambda bb, i: (bb, i, 0, 0)),
                  pl.BlockSpec((1, 1, Cin, W), lambda bb, i: (bb, i, 0, 0)),
                  pl.BlockSpec(s0.shape, lambda bb, i: (0, 0)),
                  pl.BlockSpec(s1.shape, lambda bb, i: (0, 0)),
                  pl.BlockSpec(w.shape, lambda bb, i: (0, 0, 0)),
                  pl.BlockSpec(b.shape, lambda bb, i: (0, 0))],
        out_specs=pl.BlockSpec((1, TR, Cout, Wo), lambda bb, i: (bb, i, 0, 0)),
        out_shape=jax.ShapeDtypeStruct((B, Ho, Cout, Wo), BF16),
    )(x, bot, s0, s1, w, b)


def _conv_up_body(top, cur, bot, rmat, w, b_ref, o_ref, *, TR2, Wd, Cin, Cout):
    TR, W = 2 * TR2, 2 * Wd
    dwin = jnp.concatenate([top[0], cur[0], bot[0]], axis=0)
    flat = dwin.reshape((TR2 + 2) * Cin, Wd)
    wide = _dot(flat, rmat[...]).astype(BF16).reshape(TR2 + 2, Cin, W)
    urep = jnp.repeat(wide, 2, axis=0)[1:TR + 3]
    srcs = [(_shift3(urep, W), w)]
    _accum_rows(srcs, o_ref, b_ref[...], True, TR, Cout, W, BF16)


def _conv_up(d, w, b, TR2):
    B, Hd, Cin, Wd = d.shape
    Cout = w.shape[1]
    H, W = 2 * Hd, 2 * Wd
    TR = 2 * TR2
    nb = Hd // TR2
    top, bot = _halos(d, TR2)
    rmat = (jnp.arange(Wd)[:, None] == (jnp.arange(W)[None, :] // 2)).astype(BF16)
    return pl.pallas_call(
        functools.partial(_conv_up_body, TR2=TR2, Wd=Wd, Cin=Cin, Cout=Cout),
        grid=(B, nb),
        in_specs=[pl.BlockSpec((1, 1, Cin, Wd), lambda bb, i: (bb, i, 0, 0)),
                  pl.BlockSpec((1, TR2, Cin, Wd), lambda bb, i: (bb, i, 0, 0)),
                  pl.BlockSpec((1, 1, Cin, Wd), lambda bb, i: (bb, i, 0, 0)),
                  pl.BlockSpec(rmat.shape, lambda bb, i: (0, 0)),
                  pl.BlockSpec(w.shape, lambda bb, i: (0, 0, 0)),
                  pl.BlockSpec(b.shape, lambda bb, i: (0, 0))],
        out_specs=pl.BlockSpec((1, TR, Cout, W), lambda bb, i: (bb, i, 0, 0)),
        out_shape=jax.ShapeDtypeStruct((B, H, Cout, W), BF16),
    )(top, d, bot, rmat, w, b)


def _wk(w):
    O, I = w.shape[0], w.shape[1]
    return jnp.transpose(w, (3, 0, 2, 1)).reshape(3, O, 3 * I).astype(BF16)


def kernel(image, w1, b1, w2, b2, w3, b3, w4, b4, w5, b5, w6, b6, w_out, b_out):
    x = jnp.transpose(image, (0, 2, 1, 3)).astype(BF16)
    bb = lambda b: b.reshape(-1, 1).astype(F32)
    t = _conv_s1([x], [_wk(w1)], bb(b1), True, 16)
    e1 = _conv_s1([t], [_wk(w2)], bb(b2), True, 16)
    d = _conv_s2(e1, _wk(w3), bb(b3), 16)
    d = _conv_s1([d], [_wk(w4)], bb(b4), True, 16)
    u = _conv_up(d, _wk(w5), bb(b5), 8)
    f = _conv_s1([u, e1], [_wk(w6[:, :32]), _wk(w6[:, 32:])], bb(b6), True, 16)
    y = _conv_s1([f], [_wk(w_out)], bb(b_out), True, 16, out_dtype=F32)
    return jnp.transpose(y, (0, 2, 1, 3))


# speedup vs baseline: 7.2358x; 1.1252x over previous
import functools

import jax
import jax.numpy as jnp
from jax.experimental import pallas as pl

BF16 = jnp.bfloat16
F32 = jnp.float32


def _dot(a, b):
    return jax.lax.dot_general(a, b, (((1,), (0,)), ((), ())),
                               preferred_element_type=F32)


def _shift3(win, W):
    z = jnp.zeros(win.shape[:2] + (1,), win.dtype)
    return [jnp.concatenate([z, win[:, :, :W - 1]], axis=2),
            win,
            jnp.concatenate([win[:, :, 1:], z], axis=2)]


def _accum_rows(srcs, o_ref, bias, relu, TR, Cout, W, out_dtype):
    for r in range(TR):
        acc = jnp.zeros((Cout, W), F32)
        for shifts, w in srcs:
            Cin = shifts[1].shape[1]
            for kx in range(3):
                rhs = shifts[kx][r:r + 3].reshape(3 * Cin, W)
                acc = acc + _dot(w[kx], rhs)
        acc = acc + bias
        if relu:
            acc = jnp.maximum(acc, 0.0)
        o_ref[0, r] = acc.astype(out_dtype)


def _conv_s1_body(*refs, nin, TR, W, Cout, relu, out_dtype):
    o_ref = refs[-1]
    b_ref = refs[-2]
    wins = []
    for k in range(nin):
        top, cur, bot = refs[3 * k], refs[3 * k + 1], refs[3 * k + 2]
        wins.append(jnp.concatenate([top[0], cur[0], bot[0]], axis=0))
    win = wins[0] if nin == 1 else jnp.concatenate(wins, axis=1)
    srcs = [(_shift3(win, W), refs[3 * nin])]
    _accum_rows(srcs, o_ref, b_ref[...], relu, TR, Cout, W, out_dtype)


def _halos(x, TR):
    B, H, C, W = x.shape
    z = jnp.zeros((B, 1, C, W), x.dtype)
    top = jnp.concatenate([z, x[:, TR - 1::TR][:, :-1]], axis=1)
    bot = jnp.concatenate([x[:, TR::TR], z], axis=1)
    return top, bot


def _conv_s1(xs, w, b, relu, TR, out_dtype=BF16):
    B, H, _, W = xs[0].shape
    Cout = w.shape[1]
    nb = H // TR
    args, in_specs = [], []
    for x in xs:
        Cin = x.shape[2]
        top, bot = _halos(x, TR)
        args += [top, x, bot]
        in_specs += [
            pl.BlockSpec((1, 1, Cin, W), lambda bb, i: (bb, i, 0, 0)),
            pl.BlockSpec((1, TR, Cin, W), lambda bb, i: (bb, i, 0, 0)),
            pl.BlockSpec((1, 1, Cin, W), lambda bb, i: (bb, i, 0, 0)),
        ]
    args += [w, b]
    in_specs += [pl.BlockSpec(w.shape, lambda bb, i: (0, 0, 0)),
                 pl.BlockSpec(b.shape, lambda bb, i: (0, 0))]
    return pl.pallas_call(
        functools.partial(_conv_s1_body, nin=len(xs), TR=TR, W=W, Cout=Cout,
                          relu=relu, out_dtype=out_dtype),
        grid=(B, nb),
        in_specs=in_specs,
        out_specs=pl.BlockSpec((1, TR, Cout, W), lambda bb, i: (bb, i, 0, 0)),
        out_shape=jax.ShapeDtypeStruct((B, H, Cout, W), out_dtype),
    )(*args)


def _conv_s2_body(xc, xb, s0, s1, w, b_ref, o_ref, *, TR, Wo, Cin, Cout):
    W = 2 * Wo
    z = jnp.zeros((1, Cin, W), BF16)
    win = jnp.concatenate([xc[0], xb[0], z], axis=0)
    flat = win.reshape((2 * TR + 2) * Cin, W)
    wine = _dot(flat, s0[...]).astype(BF16).reshape(2 * TR + 2, Cin, Wo)
    wino = _dot(flat, s1[...]).astype(BF16).reshape(2 * TR + 2, Cin, Wo)
    zc = jnp.zeros((2 * TR + 2, Cin, 1), BF16)
    srcs = [wine, wino, jnp.concatenate([wine[:, :, 1:], zc], axis=2)]
    for r in range(TR):
        acc = jnp.zeros((Cout, Wo), F32)
        for kx in range(3):
            rhs = srcs[kx][2 * r:2 * r + 3].reshape(3 * Cin, Wo)
            acc = acc + _dot(w[kx], rhs)
        acc = jnp.maximum(acc + b_ref[...], 0.0)
        o_ref[0, r] = acc.astype(BF16)


def _conv_s2(x, w, b, TR):
    B, H, Cin, W = x.shape
    Cout = w.shape[1]
    Ho, Wo = H // 2, W // 2
    nb = Ho // TR
    zb = jnp.zeros((B, 1, Cin, W), BF16)
    bot = jnp.concatenate([x[:, 2 * TR::2 * TR], zb], axis=1)
    cols = jnp.arange(W)[:, None]
    s0 = (cols == 2 * jnp.arange(Wo)[None, :]).astype(BF16)
    s1 = (cols == 2 * jnp.arange(Wo)[None, :] + 1).astype(BF16)
    return pl.pallas_call(
        functools.partial(_conv_s2_body, TR=TR, Wo=Wo, Cin=Cin, Cout=Cout),
        grid=(B, nb),
        in_specs=[pl.BlockSpec((1, 2 * TR, Cin, W), lambda bb, i: (bb, i, 0, 0)),
                  pl.BlockSpec((1, 1, Cin, W), lambda bb, i: (bb, i, 0, 0)),
                  pl.BlockSpec(s0.shape, lambda bb, i: (0, 0)),
                  pl.BlockSpec(s1.shape, lambda bb, i: (0, 0)),
                  pl.BlockSpec(w.shape, lambda bb, i: (0, 0, 0)),
                  pl.BlockSpec(b.shape, lambda bb, i: (0, 0))],
        out_specs=pl.BlockSpec((1, TR, Cout, Wo), lambda bb, i: (bb, i, 0, 0)),
        out_shape=jax.ShapeDtypeStruct((B, Ho, Cout, Wo), BF16),
    )(x, bot, s0, s1, w, b)


def _conv_up_body(top, cur, bot, rmat, w, b_ref, o_ref, *, TR2, Wd, Cin, Cout):
    TR, W = 2 * TR2, 2 * Wd
    dwin = jnp.concatenate([top[0], cur[0], bot[0]], axis=0)
    flat = dwin.reshape((TR2 + 2) * Cin, Wd)
    wide = _dot(flat, rmat[...]).astype(BF16).reshape(TR2 + 2, Cin, W)
    urep = jnp.repeat(wide, 2, axis=0)[1:TR + 3]
    srcs = [(_shift3(urep, W), w)]
    _accum_rows(srcs, o_ref, b_ref[...], True, TR, Cout, W, BF16)


def _conv_up(d, w, b, TR2):
    B, Hd, Cin, Wd = d.shape
    Cout = w.shape[1]
    H, W = 2 * Hd, 2 * Wd
    TR = 2 * TR2
    nb = Hd // TR2
    top, bot = _halos(d, TR2)
    rmat = (jnp.arange(Wd)[:, None] == (jnp.arange(W)[None, :] // 2)).astype(BF16)
    return pl.pallas_call(
        functools.partial(_conv_up_body, TR2=TR2, Wd=Wd, Cin=Cin, Cout=Cout),
        grid=(B, nb),
        in_specs=[pl.BlockSpec((1, 1, Cin, Wd), lambda bb, i: (bb, i, 0, 0)),
                  pl.BlockSpec((1, TR2, Cin, Wd), lambda bb, i: (bb, i, 0, 0)),
                  pl.BlockSpec((1, 1, Cin, Wd), lambda bb, i: (bb, i, 0, 0)),
                  pl.BlockSpec(rmat.shape, lambda bb, i: (0, 0)),
                  pl.BlockSpec(w.shape, lambda bb, i: (0, 0, 0)),
                  pl.BlockSpec(b.shape, lambda bb, i: (0, 0))],
        out_specs=pl.BlockSpec((1, TR, Cout, W), lambda bb, i: (bb, i, 0, 0)),
        out_shape=jax.ShapeDtypeStruct((B, H, Cout, W), BF16),
    )(top, d, bot, rmat, w, b)


def _wk(w):
    O, I = w.shape[0], w.shape[1]
    return jnp.transpose(w, (3, 0, 2, 1)).reshape(3, O, 3 * I).astype(BF16)


def kernel(image, w1, b1, w2, b2, w3, b3, w4, b4, w5, b5, w6, b6, w_out, b_out):
    x = jnp.transpose(image, (0, 2, 1, 3)).astype(BF16)
    bb = lambda b: b.reshape(-1, 1).astype(F32)
    t = _conv_s1([x], _wk(w1), bb(b1), True, 32)
    e1 = _conv_s1([t], _wk(w2), bb(b2), True, 32)
    d = _conv_s2(e1, _wk(w3), bb(b3), 32)
    d = _conv_s1([d], _wk(w4), bb(b4), True, 32)
    u = _conv_up(d, _wk(w5), bb(b5), 16)
    f = _conv_s1([u, e1], _wk(w6), bb(b6), True, 32)
    y = _conv_s1([f], _wk(w_out), bb(b_out), True, 32, out_dtype=F32)
    return jnp.transpose(y, (0, 2, 1, 3))
